# Initial kernel scaffold; baseline (speedup 1.0000x reference)
import jax
import jax.numpy as jnp
from jax.experimental import pallas as pl


def kernel(user_w, item_w, W1_0, b1_0, W2_0, b2_0, W1_1, b1_1, W2_1, b2_1, W1_2, b1_2, W2_2, b2_2, edge_index):
    raise NotImplementedError("write your pallas kernel here")


# speedup vs baseline: 7.8569x; 7.8569x over previous
import functools

import jax
import jax.numpy as jnp
from jax import lax
from jax.experimental import pallas as pl
from jax.experimental.pallas import tpu as pltpu
from jax.experimental.pallas import tpu_sc as plsc

N_USERS = 25000
N_ITEMS = 25000
N = N_USERS + N_ITEMS
E = 800000
D = 64
H = 32

NC = 2
NS = 16
CH = 128

N_PAD = 50048
STRIPE = N_PAD // NS
E_PAD = 802816
ET2 = E_PAD // (NC * NS)
ET3 = E_PAD // NS

_mesh = plsc.VectorSubcoreMesh(core_axis_name="c", subcore_axis_name="s")
_sc_params = pltpu.CompilerParams(use_tc_tiling_on_sc=False)


@functools.partial(
    pl.kernel,
    out_type=jax.ShapeDtypeStruct((2 * N_PAD, 16), jnp.float32),
    mesh=_mesh,
    compiler_params=_sc_params,
    scratch_types=[
        pltpu.VMEM((CH,), jnp.int32),
        pltpu.VMEM((CH, 16), jnp.float32),
        pltpu.VMEM_SHARED((N_PAD, 16), jnp.float32),
    ],
)
def _sc_deg(col_hbm, ones_hbm, zeros_hbm, out_hbm, cidx, ones_v, accum):
    c = lax.axis_index("c")
    s = lax.axis_index("s")
    pltpu.sync_copy(zeros_hbm, accum.at[pl.ds(s * STRIPE, STRIPE)])
    pltpu.sync_copy(ones_hbm, ones_v)
    plsc.subcore_barrier()
    base = (c * NS + s) * ET2
    @pl.loop(0, ET2 // CH)
    def _(g):
        pltpu.sync_copy(col_hbm.at[pl.ds(base + g * CH, CH)], cidx)
        pltpu.sync_copy(ones_v, accum.at[cidx], add=True)
    plsc.subcore_barrier()
    pltpu.sync_copy(accum.at[pl.ds(s * STRIPE, STRIPE)],
                    out_hbm.at[pl.ds(c * N_PAD + s * STRIPE, STRIPE)])


@functools.partial(
    pl.kernel,
    out_type=jax.ShapeDtypeStruct((2 * N_PAD, 16), jnp.float32),
    mesh=_mesh,
    compiler_params=_sc_params,
    scratch_types=[
        pltpu.VMEM((CH,), jnp.int32),
        pltpu.VMEM((CH,), jnp.int32),
        pltpu.VMEM((CH, 16), jnp.float32),
        pltpu.VMEM_SHARED((N_PAD, 16), jnp.float32),
        pltpu.SemaphoreType.DMA,
    ],
)
def _sc_t(dis2_hbm, row_hbm, col_hbm, zeros_hbm, out_hbm,
          ridx, cidx, vals, accum, sem):
    c = lax.axis_index("c")
    s = lax.axis_index("s")
    pltpu.sync_copy(zeros_hbm, accum.at[pl.ds(s * STRIPE, STRIPE)])
    plsc.subcore_barrier()
    base = (c * NS + s) * ET2
    @pl.loop(0, ET2 // CH)
    def _(g):
        pltpu.sync_copy(row_hbm.at[pl.ds(base + g * CH, CH)], ridx)
        pltpu.sync_copy(col_hbm.at[pl.ds(base + g * CH, CH)], cidx)
        pltpu.async_copy(dis2_hbm.at[ridx], vals, sem).wait()
        pltpu.sync_copy(vals, accum.at[cidx], add=True)
    plsc.subcore_barrier()
    pltpu.sync_copy(accum.at[pl.ds(s * STRIPE, STRIPE)],
                    out_hbm.at[pl.ds(c * N_PAD + s * STRIPE, STRIPE)])


@functools.partial(
    pl.kernel,
    out_type=jax.ShapeDtypeStruct((2 * N_PAD, H), jnp.float32),
    mesh=_mesh,
    compiler_params=_sc_params,
    scratch_types=[
        pltpu.VMEM((CH,), jnp.int32),
        pltpu.VMEM((CH,), jnp.int32),
        pltpu.VMEM((CH, H), jnp.float32),
        pltpu.VMEM_SHARED((N_PAD, H), jnp.float32),
        pltpu.SemaphoreType.DMA,
    ],
)
def _sc_seg(ycat_hbm, row2_hbm, col_hbm, zeros_hbm, out_hbm,
            ridx, cidx, vals, accum, sem):
    c = lax.axis_index("c")
    s = lax.axis_index("s")
    pltpu.sync_copy(zeros_hbm, accum.at[pl.ds(s * STRIPE, STRIPE)])
    plsc.subcore_barrier()
    rbase = c * E_PAD + s * ET3
    cbase = s * ET3
    @pl.loop(0, ET3 // CH)
    def _(g):
        pltpu.sync_copy(row2_hbm.at[pl.ds(rbase + g * CH, CH)], ridx)
        pltpu.sync_copy(col_hbm.at[pl.ds(cbase + g * CH, CH)], cidx)
        pltpu.async_copy(ycat_hbm.at[ridx], vals, sem).wait()
        pltpu.sync_copy(vals, accum.at[cidx], add=True)
    plsc.subcore_barrier()
    pltpu.sync_copy(accum.at[pl.ds(s * STRIPE, STRIPE)],
                    out_hbm.at[pl.ds(c * N_PAD + s * STRIPE, STRIPE)])


NB = 2000


def _tc_dis_body(p_ref, dis2_ref):
    p = p_ref[...]
    deg = p[0] + p[1]
    dis2_ref[...] = jnp.where(deg > 0, lax.rsqrt(deg), 0.0)


_tc_dis = pl.pallas_call(
    _tc_dis_body,
    grid=(N // NB,),
    in_specs=[pl.BlockSpec((2, NB, 16), lambda i: (0, i, 0))],
    out_specs=pl.BlockSpec((NB, 16), lambda i: (i, 0)),
    out_shape=jax.ShapeDtypeStruct((N, 16), jnp.float32),
)


def _tc_prep_body(p_ref, dis2_ref, x0_ref, s16_ref, y0_ref, y1_ref):
    p = p_ref[...]
    t = p[0] + p[1]
    d16 = dis2_ref[...]
    s16_ref[...] = d16 * t
    y = d16[:, 0:1] * x0_ref[...]
    y0_ref[...] = y[:, :H]
    y1_ref[...] = y[:, H:]


_tc_prep = pl.pallas_call(
    _tc_prep_body,
    grid=(N // NB,),
    in_specs=[
        pl.BlockSpec((2, NB, 16), lambda i: (0, i, 0)),
        pl.BlockSpec((NB, 16), lambda i: (i, 0)),
        pl.BlockSpec((NB, D), lambda i: (i, 0)),
    ],
    out_specs=[
        pl.BlockSpec((NB, 16), lambda i: (i, 0)),
        pl.BlockSpec((NB, H), lambda i: (i, 0)),
        pl.BlockSpec((NB, H), lambda i: (i, 0)),
    ],
    out_shape=[
        jax.ShapeDtypeStruct((N, 16), jnp.float32),
        jax.ShapeDtypeStruct((N, H), jnp.float32),
        jax.ShapeDtypeStruct((N, H), jnp.float32),
    ],
)


def _tc_layer_body(g_ref, dis2_ref, x_ref, s16_ref, w1_ref, w2_ref, b_ref,
                   xn_ref, y0_ref, y1_ref):
    g = g_ref[...]
    graw = jnp.concatenate([g[0], g[1]], axis=1)
    d1 = dis2_ref[...][:, 0:1]
    a = d1 * graw
    x = x_ref[...]
    acc = lax.dot_general(a, w1_ref[...], (((1,), (1,)), ((), ())),
                          preferred_element_type=jnp.float32)
    acc = acc + lax.dot_general(x * a, w2_ref[...], (((1,), (1,)), ((), ())),
                                preferred_element_type=jnp.float32)
    acc = acc + s16_ref[...][:, 0:1] * b_ref[...]
    xn = jnp.where(acc >= 0, acc, 0.01 * acc)
    xn_ref[...] = xn
    y = d1 * xn
    y0_ref[...] = y[:, :H]
    y1_ref[...] = y[:, H:]


_tc_layer = pl.pallas_call(
    _tc_layer_body,
    grid=(N // NB,),
    in_specs=[
        pl.BlockSpec((2, NB, H), lambda i: (0, i, 0)),
        pl.BlockSpec((NB, 16), lambda i: (i, 0)),
        pl.BlockSpec((NB, D), lambda i: (i, 0)),
        pl.BlockSpec((NB, 16), lambda i: (i, 0)),
        pl.BlockSpec((D, D), lambda i: (0, 0)),
        pl.BlockSpec((D, D), lambda i: (0, 0)),
        pl.BlockSpec((1, D), lambda i: (0, 0)),
    ],
    out_specs=[
        pl.BlockSpec((NB, D), lambda i: (i, 0)),
        pl.BlockSpec((NB, H), lambda i: (i, 0)),
        pl.BlockSpec((NB, H), lambda i: (i, 0)),
    ],
    out_shape=[
        jax.ShapeDtypeStruct((N, D), jnp.float32),
        jax.ShapeDtypeStruct((N, H), jnp.float32),
        jax.ShapeDtypeStruct((N, H), jnp.float32),
    ],
)


def kernel(user_w, item_w, W1_0, b1_0, W2_0, b2_0, W1_1, b1_1, W2_1, b2_1,
           W1_2, b1_2, W2_2, b2_2, edge_index):
    row = edge_index[0].astype(jnp.int32)
    col = edge_index[1].astype(jnp.int32)
    pad = E_PAD - E
    colp = jnp.concatenate([col, jnp.full((pad,), N, jnp.int32)])
    rowp = jnp.concatenate([row, jnp.zeros((pad,), jnp.int32)])
    row2 = jnp.concatenate([rowp, rowp + N])

    ones16 = jnp.ones((CH, 16), jnp.float32)
    zeros16 = jnp.zeros((STRIPE, 16), jnp.float32)
    zeros32 = jnp.zeros((STRIPE, H), jnp.float32)
    x0 = jnp.concatenate([user_w, item_w], axis=0)

    dparts = _sc_deg(colp, ones16, zeros16).reshape(2, N_PAD, 16)
    dis2 = _tc_dis(dparts)
    tparts = _sc_t(dis2, rowp, colp, zeros16).reshape(2, N_PAD, 16)
    s16, y0, y1 = _tc_prep(tparts, dis2, x0)

    params = [(W1_0, b1_0, W2_0, b2_0), (W1_1, b1_1, W2_1, b2_1),
              (W1_2, b1_2, W2_2, b2_2)]
    embs = [x0]
    x = x0
    for (W1, b1, W2, b2) in params:
        ycat = jnp.concatenate([y0, y1], axis=0)
        gparts = _sc_seg(ycat, row2, colp, zeros32).reshape(2, N_PAD, H)
        bsum = (b1 + b2).reshape(1, D)
        x, y0, y1 = _tc_layer(gparts, dis2, x, s16, W1, W2, bsum)
        embs.append(x)

    out = jnp.concatenate(embs, axis=1)
    return out[:N_USERS], out[N_USERS:]


# speedup vs baseline: 16.4583x; 2.0948x over previous
import functools

import jax
import jax.numpy as jnp
from jax import lax
from jax.experimental import pallas as pl
from jax.experimental.pallas import tpu as pltpu
from jax.experimental.pallas import tpu_sc as plsc

N_USERS = 25000
N_ITEMS = 25000
N = N_USERS + N_ITEMS
E = 800000
D = 64
H = 32

NC = 2
NS = 16
CH = 128

N_PAD = 50048
STRIPE = N_PAD // NS
E_PAD = 802816
ET2 = E_PAD // (NC * NS)
ET3 = E_PAD // NS

_mesh = plsc.VectorSubcoreMesh(core_axis_name="c", subcore_axis_name="s")
_sc_params = pltpu.CompilerParams(use_tc_tiling_on_sc=False)


NGC2 = ET2 // CH
NGC3 = ET3 // CH
RING = 4


@functools.partial(
    pl.kernel,
    out_type=jax.ShapeDtypeStruct((2 * N_PAD, 16), jnp.float32),
    mesh=_mesh,
    compiler_params=_sc_params,
    scratch_types=[
        pltpu.VMEM((2, CH), jnp.int32),
        pltpu.VMEM((2, CH), jnp.int32),
        pltpu.VMEM((2, CH), jnp.int32),
        pltpu.VMEM((2, CH), jnp.int32),
        pltpu.VMEM((CH, 16), jnp.float32),
        pltpu.VMEM_SHARED((N_PAD, 16), jnp.float32),
        pltpu.SemaphoreType.DMA,
        pltpu.SemaphoreType.DMA,
    ],
)
def _sc_deg(rc_hbm, ones_hbm, zeros_hbm, out_hbm,
            i0, i1, i2, i3, ones_v, accum, isem, ssem):
    c = lax.axis_index("c")
    s = lax.axis_index("s")
    ibufs = (i0, i1, i2, i3)
    base = (c * NS + s) * NGC2
    for j in range(RING):
        pltpu.async_copy(rc_hbm.at[base + j], ibufs[j], isem)
    pltpu.sync_copy(ones_hbm, ones_v)
    pltpu.sync_copy(zeros_hbm, accum.at[pl.ds(s * STRIPE, STRIPE)])
    plsc.subcore_barrier()

    @pl.loop(0, NGC2 // RING)
    def _(u):
        for j in range(RING):
            g = u * RING + j
            ib = ibufs[j]

            @pl.when(g >= 2)
            def _():
                pltpu.make_async_copy(ones_v, accum.at[ib.at[1]], ssem).wait()

            @pl.when(jnp.logical_and(g >= 2, g + 2 < NGC2))
            def _():
                pltpu.async_copy(rc_hbm.at[base + g + 2], ibufs[(j + 2) % RING],
                                 isem)
            pltpu.make_async_copy(rc_hbm.at[base + g], ib, isem).wait()
            pltpu.async_copy(ones_v, accum.at[ib.at[1]], ssem, add=True)

    pltpu.make_async_copy(ones_v, accum.at[i0.at[1]], ssem).wait()
    pltpu.make_async_copy(ones_v, accum.at[i0.at[1]], ssem).wait()
    plsc.subcore_barrier()
    pltpu.sync_copy(accum.at[pl.ds(s * STRIPE, STRIPE)],
                    out_hbm.at[pl.ds(c * N_PAD + s * STRIPE, STRIPE)])


def _pipelined_gather_scatter(table_hbm, rc_hbm, base, ibufs, vbufs,
                              accum, isem, gsem, ssem, ngc):
    for j in range(RING):
        pltpu.async_copy(rc_hbm.at[base + j], ibufs[j], isem)
    pltpu.make_async_copy(rc_hbm.at[base], ibufs[0], isem).wait()
    pltpu.async_copy(table_hbm.at[ibufs[0].at[0]], vbufs[0], gsem)

    @pl.loop(0, ngc // RING)
    def _(u):
        for j in range(RING):
            g = u * RING + j
            ib, vb = ibufs[j], vbufs[j]
            jn = (j + 1) % RING

            @pl.when(g >= 2)
            def _():
                pltpu.make_async_copy(vb, accum.at[ib.at[1]], ssem).wait()

            @pl.when(jnp.logical_and(g >= 2, g + 2 < ngc))
            def _():
                pltpu.async_copy(rc_hbm.at[base + g + 2], ibufs[(j + 2) % RING],
                                 isem)

            @pl.when(g + 1 < ngc)
            def _():
                pltpu.make_async_copy(rc_hbm.at[base + g + 1], ibufs[jn],
                                      isem).wait()
                pltpu.async_copy(table_hbm.at[ibufs[jn].at[0]], vbufs[jn], gsem)
            pltpu.make_async_copy(table_hbm.at[ib.at[0]], vb, gsem).wait()
            pltpu.async_copy(vb, accum.at[ib.at[1]], ssem, add=True)

    pltpu.make_async_copy(vbufs[0], accum.at[ibufs[0].at[1]], ssem).wait()
    pltpu.make_async_copy(vbufs[0], accum.at[ibufs[0].at[1]], ssem).wait()


def _make_seg_kernel(width, ngc):
    @functools.partial(
        pl.kernel,
        out_type=jax.ShapeDtypeStruct((2 * N_PAD, width), jnp.float32),
        mesh=_mesh,
        compiler_params=_sc_params,
        scratch_types=[
            pltpu.VMEM((2, CH), jnp.int32),
            pltpu.VMEM((2, CH), jnp.int32),
            pltpu.VMEM((2, CH), jnp.int32),
            pltpu.VMEM((2, CH), jnp.int32),
            pltpu.VMEM((CH, width), jnp.float32),
            pltpu.VMEM((CH, width), jnp.float32),
            pltpu.VMEM((CH, width), jnp.float32),
            pltpu.VMEM((CH, width), jnp.float32),
            pltpu.VMEM_SHARED((N_PAD, width), jnp.float32),
            pltpu.SemaphoreType.DMA,
            pltpu.SemaphoreType.DMA,
            pltpu.SemaphoreType.DMA,
        ],
    )
    def seg(table_hbm, rc_hbm, zeros_hbm, out_hbm,
            i0, i1, i2, i3, v0, v1, v2, v3, accum, isem, gsem, ssem):
        c = lax.axis_index("c")
        s = lax.axis_index("s")
        nchunk_rows = rc_hbm.shape[0]
        two_core_scan = nchunk_rows == 2 * (E_PAD // CH)
        if two_core_scan:
            base = c * (E_PAD // CH) + s * ngc
        else:
            base = (c * NS + s) * ngc
        pltpu.sync_copy(zeros_hbm, accum.at[pl.ds(s * STRIPE, STRIPE)])
        plsc.subcore_barrier()
        _pipelined_gather_scatter(table_hbm, rc_hbm, base,
                                  (i0, i1, i2, i3), (v0, v1, v2, v3),
                                  accum, isem, gsem, ssem, ngc)
        plsc.subcore_barrier()
        pltpu.sync_copy(accum.at[pl.ds(s * STRIPE, STRIPE)],
                        out_hbm.at[pl.ds(c * N_PAD + s * STRIPE, STRIPE)])

    return seg


_sc_t = _make_seg_kernel(16, NGC2)
_sc_seg = _make_seg_kernel(H, NGC3)


NB = 2000


def _tc_dis_body(p_ref, dis2_ref):
    p = p_ref[...]
    deg = p[0] + p[1]
    dis2_ref[...] = jnp.where(deg > 0, lax.rsqrt(deg), 0.0)


_tc_dis = pl.pallas_call(
    _tc_dis_body,
    grid=(N // NB,),
    in_specs=[pl.BlockSpec((2, NB, 16), lambda i: (0, i, 0))],
    out_specs=pl.BlockSpec((NB, 16), lambda i: (i, 0)),
    out_shape=jax.ShapeDtypeStruct((N, 16), jnp.float32),
)


def _tc_prep_body(p_ref, dis2_ref, x0_ref, s16_ref, y0_ref, y1_ref):
    p = p_ref[...]
    t = p[0] + p[1]
    d16 = dis2_ref[...]
    s16_ref[...] = d16 * t
    y = d16[:, 0:1] * x0_ref[...]
    y0_ref[...] = y[:, :H]
    y1_ref[...] = y[:, H:]


_tc_prep = pl.pallas_call(
    _tc_prep_body,
    grid=(N // NB,),
    in_specs=[
        pl.BlockSpec((2, NB, 16), lambda i: (0, i, 0)),
        pl.BlockSpec((NB, 16), lambda i: (i, 0)),
        pl.BlockSpec((NB, D), lambda i: (i, 0)),
    ],
    out_specs=[
        pl.BlockSpec((NB, 16), lambda i: (i, 0)),
        pl.BlockSpec((NB, H), lambda i: (i, 0)),
        pl.BlockSpec((NB, H), lambda i: (i, 0)),
    ],
    out_shape=[
        jax.ShapeDtypeStruct((N, 16), jnp.float32),
        jax.ShapeDtypeStruct((N, H), jnp.float32),
        jax.ShapeDtypeStruct((N, H), jnp.float32),
    ],
)


def _tc_layer_body(g_ref, dis2_ref, x_ref, s16_ref, w1_ref, w2_ref, b_ref,
                   xn_ref, y0_ref, y1_ref):
    g = g_ref[...]
    graw = jnp.concatenate([g[0], g[1]], axis=1)
    d1 = dis2_ref[...][:, 0:1]
    a = d1 * graw
    x = x_ref[...]
    acc = lax.dot_general(a, w1_ref[...], (((1,), (1,)), ((), ())),
                          preferred_element_type=jnp.float32)
    acc = acc + lax.dot_general(x * a, w2_ref[...], (((1,), (1,)), ((), ())),
                                preferred_element_type=jnp.float32)
    acc = acc + s16_ref[...][:, 0:1] * b_ref[...]
    xn = jnp.where(acc >= 0, acc, 0.01 * acc)
    xn_ref[...] = xn
    y = d1 * xn
    y0_ref[...] = y[:, :H]
    y1_ref[...] = y[:, H:]


_tc_layer = pl.pallas_call(
    _tc_layer_body,
    grid=(N // NB,),
    in_specs=[
        pl.BlockSpec((2, NB, H), lambda i: (0, i, 0)),
        pl.BlockSpec((NB, 16), lambda i: (i, 0)),
        pl.BlockSpec((NB, D), lambda i: (i, 0)),
        pl.BlockSpec((NB, 16), lambda i: (i, 0)),
        pl.BlockSpec((D, D), lambda i: (0, 0)),
        pl.BlockSpec((D, D), lambda i: (0, 0)),
        pl.BlockSpec((1, D), lambda i: (0, 0)),
    ],
    out_specs=[
        pl.BlockSpec((NB, D), lambda i: (i, 0)),
        pl.BlockSpec((NB, H), lambda i: (i, 0)),
        pl.BlockSpec((NB, H), lambda i: (i, 0)),
    ],
    out_shape=[
        jax.ShapeDtypeStruct((N, D), jnp.float32),
        jax.ShapeDtypeStruct((N, H), jnp.float32),
        jax.ShapeDtypeStruct((N, H), jnp.float32),
    ],
)


def kernel(user_w, item_w, W1_0, b1_0, W2_0, b2_0, W1_1, b1_1, W2_1, b2_1,
           W1_2, b1_2, W2_2, b2_2, edge_index):
    row = edge_index[0].astype(jnp.int32)
    col = edge_index[1].astype(jnp.int32)
    pad = E_PAD - E
    colp = jnp.concatenate([col, jnp.full((pad,), N, jnp.int32)])
    rowp = jnp.concatenate([row, jnp.zeros((pad,), jnp.int32)])
    col2d = colp.reshape(E_PAD // CH, CH)
    row2d = rowp.reshape(E_PAD // CH, CH)
    rc2 = jnp.stack([row2d, col2d], axis=1)
    rc3 = jnp.concatenate([rc2, jnp.stack([row2d + N, col2d], axis=1)])

    ones16 = jnp.ones((CH, 16), jnp.float32)
    zeros16 = jnp.zeros((STRIPE, 16), jnp.float32)
    zeros32 = jnp.zeros((STRIPE, H), jnp.float32)
    x0 = jnp.concatenate([user_w, item_w], axis=0)

    dparts = _sc_deg(rc2, ones16, zeros16).reshape(2, N_PAD, 16)
    dis2 = _tc_dis(dparts)
    tparts = _sc_t(dis2, rc2, zeros16).reshape(2, N_PAD, 16)
    s16, y0, y1 = _tc_prep(tparts, dis2, x0)

    params = [(W1_0, b1_0, W2_0, b2_0), (W1_1, b1_1, W2_1, b2_1),
              (W1_2, b1_2, W2_2, b2_2)]
    embs = [x0]
    x = x0
    for (W1, b1, W2, b2) in params:
        ycat = jnp.concatenate([y0, y1], axis=0)
        gparts = _sc_seg(ycat, rc3, zeros32).reshape(2, N_PAD, H)
        bsum = (b1 + b2).reshape(1, D)
        x, y0, y1 = _tc_layer(gparts, dis2, x, s16, W1, W2, bsum)
        embs.append(x)

    out = jnp.concatenate(embs, axis=1)
    return out[:N_USERS], out[N_USERS:]
